# baseline (speedup 1.0000x reference)
import jax
import jax.numpy as jnp
from jax import lax
from jax.experimental import pallas as pl
from jax.experimental.pallas import tpu as pltpu

_N = 4096
_BS = 512
_NB = _N // _BS
_C = 32


def _dot16(a16, b16):
    return lax.dot_general(a16, b16, (((1,), (0,)), ((), ())),
                           preferred_element_type=jnp.float32)


def _body(L0, L1, L2, x0, x1, x2,
          w00, w01, w10, w11, w20, w21,
          lw0, lb0, lw1, lb1, lw2, lb2,
          out_ref, L16, sbuf, y1_ref, h0_ref, h1_ref, sem):
    Ls = (L0, L1, L2)
    xs = (x0, x1, x2)
    wAs = (w00, w10, w20)
    wBs = (w01, w11, w21)
    lws = (lw0, lw1, lw2)
    lbs = (lb0, lb1, lb2)

    def copy_blk(r, k, slot):
        return pltpu.make_async_copy(
            Ls[r].at[pl.ds(k * _BS, _BS), :], sbuf.at[slot], sem.at[slot])

    def prep_layer1(r):
        h0 = jnp.dot(xs[r][:], wAs[r][:], preferred_element_type=jnp.float32)
        h0_ref[:] = h0.astype(jnp.bfloat16)

    def layer1_block(r, k, next_r):
        slot = lax.rem(k, 2)
        copy_blk(r, k, slot).wait()
        blk16 = sbuf[slot].astype(jnp.bfloat16)
        L16[pl.ds(k * _BS, _BS), :] = blk16
        y1_ref[pl.ds(k * _BS, _BS), :] = jnp.maximum(
            _dot16(blk16, h0_ref[:]), 0.0)

        @pl.when(k + 2 < _NB)
        def _():
            copy_blk(r, k + 2, slot).start()

        if next_r is not None:
            @pl.when(k == _NB - 2)
            def _():
                copy_blk(next_r, 0, slot).start()

            @pl.when(k == _NB - 1)
            def _():
                copy_blk(next_r, 1, slot).start()

    def layer2_block(k, acc):
        y2 = jnp.maximum(_dot16(L16[pl.ds(k * _BS, _BS), :], h1_ref[:]), 0.0)
        return acc + jnp.sum(y2, axis=0, keepdims=True)

    copy_blk(0, 0, 0).start()
    copy_blk(0, 1, 1).start()
    prep_layer1(0)

    def m0(k, c):
        layer1_block(0, k, 1)
        return c
    lax.fori_loop(0, _NB, m0, 0, unroll=False)

    outs = []
    for r in (1, 2):
        h1_ref[:] = jnp.dot(y1_ref[:], wBs[r - 1][:],
                            preferred_element_type=jnp.float32
                            ).astype(jnp.bfloat16)
        prep_layer1(r)

        def m_mid(k, acc, r=r):
            acc = layer2_block(k, acc)
            layer1_block(r, k, r + 1 if r < 2 else None)
            return acc
        acc = lax.fori_loop(0, _NB, m_mid, jnp.zeros((1, _C), jnp.float32),
                            unroll=False)
        outs.append(jnp.dot(acc * (1.0 / _N), lws[r - 1][:],
                            preferred_element_type=jnp.float32)
                    + lbs[r - 1][:])

    h1_ref[:] = jnp.dot(y1_ref[:], wBs[2][:],
                        preferred_element_type=jnp.float32).astype(jnp.bfloat16)
    acc = lax.fori_loop(0, _NB, layer2_block,
                        jnp.zeros((1, _C), jnp.float32), unroll=False)
    outs.append(jnp.dot(acc * (1.0 / _N), lws[2][:],
                        preferred_element_type=jnp.float32) + lbs[2][:])

    out_ref[:] = outs[0] + outs[1] + outs[2]


def kernel(x_0, x_1, x_2, laplacian_0, laplacian_1, laplacian_2,
           W0_l0, W1_l0, W2_l0, W0_l1, W1_l1, W2_l1,
           lin0_w, lin0_b, lin1_w, lin1_b, lin2_w, lin2_b):
    ncls = lin0_w.shape[1]
    hbm = pl.BlockSpec(memory_space=pltpu.MemorySpace.HBM)
    vmem = pl.BlockSpec(memory_space=pltpu.VMEM)
    out = pl.pallas_call(
        _body,
        out_shape=jax.ShapeDtypeStruct((1, ncls), jnp.float32),
        in_specs=[hbm, hbm, hbm] + [vmem] * 15,
        out_specs=vmem,
        scratch_shapes=[
            pltpu.VMEM((_N, _N), jnp.bfloat16),
            pltpu.VMEM((2, _BS, _N), jnp.float32),
            pltpu.VMEM((_N, _C), jnp.float32),
            pltpu.VMEM((_N, _C), jnp.bfloat16),
            pltpu.VMEM((_N, _C), jnp.bfloat16),
            pltpu.SemaphoreType.DMA((2,)),
        ],
        compiler_params=pltpu.CompilerParams(
            vmem_limit_bytes=62 * 1024 * 1024),
    )(laplacian_0, laplacian_1, laplacian_2, x_0, x_1, x_2,
      W0_l0, W0_l1, W1_l0, W1_l1, W2_l0, W2_l1,
      lin0_w, lin0_b.reshape(1, ncls), lin1_w, lin1_b.reshape(1, ncls),
      lin2_w, lin2_b.reshape(1, ncls))
    return out.reshape(-1)


# speedup vs baseline: 1.2243x; 1.2243x over previous
import jax
import jax.numpy as jnp
from jax import lax
from jax.experimental import pallas as pl
from jax.experimental.pallas import tpu as pltpu

_N = 4096
_BS = 256
_NB = _N // _BS
_C = 32


def _dot16(a16, b16):
    return lax.dot_general(a16, b16, (((1,), (0,)), ((), ())),
                           preferred_element_type=jnp.float32)


def _body(L0, L1, L2, x0, x1, x2,
          w00, w01, w10, w11, w20, w21,
          lw0, lb0, lw1, lb1, lw2, lb2,
          out_ref, L16, sbuf, y1_ref, h0_ref, h1_ref, sem):
    Ls = (L0, L1, L2)
    xs = (x0, x1, x2)
    wAs = (w00, w10, w20)
    wBs = (w01, w11, w21)
    lws = (lw0, lw1, lw2)
    lbs = (lb0, lb1, lb2)

    def copy_blk(r, k, slot):
        return pltpu.make_async_copy(
            Ls[r].at[pl.ds(k * _BS, _BS), :], sbuf.at[slot], sem.at[slot])

    def prep_layer1(r):
        h0 = jnp.dot(xs[r][:], wAs[r][:], preferred_element_type=jnp.float32)
        h0_ref[:] = h0.astype(jnp.bfloat16)

    def layer1_block(r, k, next_r):
        slot = lax.rem(k, 2)
        copy_blk(r, k, slot).wait()
        blk16 = sbuf[slot].astype(jnp.bfloat16)
        L16[pl.ds(k * _BS, _BS), :] = blk16
        y1_ref[pl.ds(k * _BS, _BS), :] = jnp.maximum(
            _dot16(blk16, h0_ref[:]), 0.0)

        @pl.when(k + 2 < _NB)
        def _():
            copy_blk(r, k + 2, slot).start()

        if next_r is not None:
            @pl.when(k == _NB - 2)
            def _():
                copy_blk(next_r, 0, slot).start()

            @pl.when(k == _NB - 1)
            def _():
                copy_blk(next_r, 1, slot).start()

    def layer2_block(k, acc):
        y2 = jnp.maximum(_dot16(L16[pl.ds(k * _BS, _BS), :], h1_ref[:]), 0.0)
        return acc + jnp.sum(y2, axis=0, keepdims=True)

    copy_blk(0, 0, 0).start()
    copy_blk(0, 1, 1).start()
    prep_layer1(0)

    def m0(k, c):
        layer1_block(0, k, 1)
        return c
    lax.fori_loop(0, _NB, m0, 0, unroll=False)

    outs = []
    for r in (1, 2):
        h1_ref[:] = jnp.dot(y1_ref[:], wBs[r - 1][:],
                            preferred_element_type=jnp.float32
                            ).astype(jnp.bfloat16)
        prep_layer1(r)

        def m_mid(k, acc, r=r):
            acc = layer2_block(k, acc)
            layer1_block(r, k, r + 1 if r < 2 else None)
            return acc
        acc = lax.fori_loop(0, _NB, m_mid, jnp.zeros((1, _C), jnp.float32),
                            unroll=False)
        outs.append(jnp.dot(acc * (1.0 / _N), lws[r - 1][:],
                            preferred_element_type=jnp.float32)
                    + lbs[r - 1][:])

    h1_ref[:] = jnp.dot(y1_ref[:], wBs[2][:],
                        preferred_element_type=jnp.float32).astype(jnp.bfloat16)
    acc = lax.fori_loop(0, _NB, layer2_block,
                        jnp.zeros((1, _C), jnp.float32), unroll=False)
    outs.append(jnp.dot(acc * (1.0 / _N), lws[2][:],
                        preferred_element_type=jnp.float32) + lbs[2][:])

    out_ref[:] = outs[0] + outs[1] + outs[2]


def kernel(x_0, x_1, x_2, laplacian_0, laplacian_1, laplacian_2,
           W0_l0, W1_l0, W2_l0, W0_l1, W1_l1, W2_l1,
           lin0_w, lin0_b, lin1_w, lin1_b, lin2_w, lin2_b):
    ncls = lin0_w.shape[1]
    hbm = pl.BlockSpec(memory_space=pltpu.MemorySpace.HBM)
    vmem = pl.BlockSpec(memory_space=pltpu.VMEM)
    out = pl.pallas_call(
        _body,
        out_shape=jax.ShapeDtypeStruct((1, ncls), jnp.float32),
        in_specs=[hbm, hbm, hbm] + [vmem] * 15,
        out_specs=vmem,
        scratch_shapes=[
            pltpu.VMEM((_N, _N), jnp.bfloat16),
            pltpu.VMEM((2, _BS, _N), jnp.float32),
            pltpu.VMEM((_N, _C), jnp.float32),
            pltpu.VMEM((_N, _C), jnp.bfloat16),
            pltpu.VMEM((_N, _C), jnp.bfloat16),
            pltpu.SemaphoreType.DMA((2,)),
        ],
        compiler_params=pltpu.CompilerParams(
            vmem_limit_bytes=62 * 1024 * 1024),
    )(laplacian_0, laplacian_1, laplacian_2, x_0, x_1, x_2,
      W0_l0, W0_l1, W1_l0, W1_l1, W2_l0, W2_l1,
      lin0_w, lin0_b.reshape(1, ncls), lin1_w, lin1_b.reshape(1, ncls),
      lin2_w, lin2_b.reshape(1, ncls))
    return out.reshape(-1)


# speedup vs baseline: 1.2747x; 1.0412x over previous
import jax
import jax.numpy as jnp
from jax import lax
from jax.experimental import pallas as pl
from jax.experimental.pallas import tpu as pltpu

_N = 4096
_BS = 256
_NB = _N // _BS
_C = 32


def _dot16(a16, b16):
    return lax.dot_general(a16, b16, (((1,), (0,)), ((), ())),
                           preferred_element_type=jnp.float32)


def _body(L0, L1, L2, x0, x1, x2,
          w00, w01, w10, w11, w20, w21,
          lw0, lb0, lw1, lb1, lw2, lb2,
          out_ref, L16, sbuf, y1_ref, h0_ref, h1_ref, sem):
    Ls = (L0, L1, L2)
    xs = (x0, x1, x2)
    wAs = (w00, w10, w20)
    wBs = (w01, w11, w21)
    lws = (lw0, lw1, lw2)
    lbs = (lb0, lb1, lb2)

    def copy_blk(r, k, slot):
        return pltpu.make_async_copy(
            Ls[r].at[pl.ds(k * _BS, _BS), :], sbuf.at[slot], sem.at[slot])

    def prep_layer1(r):
        h0 = jnp.dot(xs[r][:], wAs[r][:], preferred_element_type=jnp.float32)
        h0_ref[:] = h0.astype(jnp.bfloat16)

    def layer1_block(r, k, next_r):
        slot = lax.rem(k, 2)
        copy_blk(r, k, slot).wait()
        blk16 = sbuf[slot].astype(jnp.bfloat16)
        L16[pl.ds(k * _BS, _BS), :] = blk16
        y1_ref[pl.ds(k * _BS, _BS), :] = jnp.maximum(
            _dot16(blk16, h0_ref[:]), 0.0)

        @pl.when(k + 2 < _NB)
        def _():
            copy_blk(r, k + 2, slot).start()

        if next_r is not None:
            @pl.when(k == _NB - 2)
            def _():
                copy_blk(next_r, 0, slot).start()

            @pl.when(k == _NB - 1)
            def _():
                copy_blk(next_r, 1, slot).start()

    def layer2_block(k, acc):
        y2 = jnp.maximum(_dot16(L16[pl.ds(k * _BS, _BS), :], h1_ref[:]), 0.0)
        return acc + jnp.sum(y2, axis=0, keepdims=True)

    copy_blk(0, 0, 0).start()
    copy_blk(0, 1, 1).start()
    prep_layer1(0)

    def m0(k, c):
        layer1_block(0, k, 1)
        return c
    lax.fori_loop(0, _NB, m0, 0, unroll=2)

    outs = []
    for r in (1, 2):
        h1_ref[:] = jnp.dot(y1_ref[:], wBs[r - 1][:],
                            preferred_element_type=jnp.float32
                            ).astype(jnp.bfloat16)
        prep_layer1(r)

        def m_mid(k, acc, r=r):
            acc = layer2_block(k, acc)
            layer1_block(r, k, r + 1 if r < 2 else None)
            return acc
        acc = lax.fori_loop(0, _NB, m_mid, jnp.zeros((1, _C), jnp.float32),
                            unroll=2)
        outs.append(jnp.dot(acc * (1.0 / _N), lws[r - 1][:],
                            preferred_element_type=jnp.float32)
                    + lbs[r - 1][:])

    h1_ref[:] = jnp.dot(y1_ref[:], wBs[2][:],
                        preferred_element_type=jnp.float32).astype(jnp.bfloat16)
    acc = lax.fori_loop(0, _NB, layer2_block,
                        jnp.zeros((1, _C), jnp.float32), unroll=2)
    outs.append(jnp.dot(acc * (1.0 / _N), lws[2][:],
                        preferred_element_type=jnp.float32) + lbs[2][:])

    out_ref[:] = outs[0] + outs[1] + outs[2]


def kernel(x_0, x_1, x_2, laplacian_0, laplacian_1, laplacian_2,
           W0_l0, W1_l0, W2_l0, W0_l1, W1_l1, W2_l1,
           lin0_w, lin0_b, lin1_w, lin1_b, lin2_w, lin2_b):
    ncls = lin0_w.shape[1]
    hbm = pl.BlockSpec(memory_space=pltpu.MemorySpace.HBM)
    vmem = pl.BlockSpec(memory_space=pltpu.VMEM)
    out = pl.pallas_call(
        _body,
        out_shape=jax.ShapeDtypeStruct((1, ncls), jnp.float32),
        in_specs=[hbm, hbm, hbm] + [vmem] * 15,
        out_specs=vmem,
        scratch_shapes=[
            pltpu.VMEM((_N, _N), jnp.bfloat16),
            pltpu.VMEM((2, _BS, _N), jnp.float32),
            pltpu.VMEM((_N, _C), jnp.float32),
            pltpu.VMEM((_N, _C), jnp.bfloat16),
            pltpu.VMEM((_N, _C), jnp.bfloat16),
            pltpu.SemaphoreType.DMA((2,)),
        ],
        compiler_params=pltpu.CompilerParams(
            vmem_limit_bytes=62 * 1024 * 1024),
    )(laplacian_0, laplacian_1, laplacian_2, x_0, x_1, x_2,
      W0_l0, W0_l1, W1_l0, W1_l1, W2_l0, W2_l1,
      lin0_w, lin0_b.reshape(1, ncls), lin1_w, lin1_b.reshape(1, ncls),
      lin2_w, lin2_b.reshape(1, ncls))
    return out.reshape(-1)


# speedup vs baseline: 1.2872x; 1.0098x over previous
import jax
import jax.numpy as jnp
from jax import lax
from jax.experimental import pallas as pl
from jax.experimental.pallas import tpu as pltpu

_N = 4096
_BS = 256
_NB = _N // _BS
_C = 32


def _dot16(a16, b16):
    return lax.dot_general(a16, b16, (((1,), (0,)), ((), ())),
                           preferred_element_type=jnp.float32)


def _body(L0, L1, L2, x0, x1, x2,
          w00, w01, w10, w11, w20, w21,
          lw0, lb0, lw1, lb1, lw2, lb2,
          out_ref, L16, sbuf, y1_ref, h0_ref, h1_ref, sem):
    Ls = (L0, L1, L2)
    xs = (x0, x1, x2)
    wAs = (w00, w10, w20)
    wBs = (w01, w11, w21)
    lws = (lw0, lw1, lw2)
    lbs = (lb0, lb1, lb2)

    def copy_blk(r, k, slot):
        return pltpu.make_async_copy(
            Ls[r].at[pl.ds(k * _BS, _BS), :], sbuf.at[slot], sem.at[slot])

    def prep_layer1(r):
        h0 = jnp.dot(xs[r][:], wAs[r][:], preferred_element_type=jnp.float32)
        h0_ref[:] = h0.astype(jnp.bfloat16)

    def layer1_block(r, k, next_r):
        slot = lax.rem(k, 2)
        copy_blk(r, k, slot).wait()
        blk16 = sbuf[slot].astype(jnp.bfloat16)
        L16[pl.ds(k * _BS, _BS), :] = blk16
        y1_ref[pl.ds(k * _BS, _BS), :] = jnp.maximum(
            _dot16(blk16, h0_ref[:]), 0.0)

        @pl.when(k + 2 < _NB)
        def _():
            copy_blk(r, k + 2, slot).start()

        if next_r is not None:
            @pl.when(k == _NB - 2)
            def _():
                copy_blk(next_r, 0, slot).start()

            @pl.when(k == _NB - 1)
            def _():
                copy_blk(next_r, 1, slot).start()

    def layer2_block(k, acc):
        y2 = jnp.maximum(_dot16(L16[pl.ds(k * _BS, _BS), :], h1_ref[:]), 0.0)
        return acc + jnp.sum(y2, axis=0, keepdims=True)

    copy_blk(0, 0, 0).start()
    copy_blk(0, 1, 1).start()
    prep_layer1(0)

    def m0(k, c):
        layer1_block(0, k, 1)
        return c
    lax.fori_loop(0, _NB, m0, 0, unroll=4)

    outs = []
    for r in (1, 2):
        h1_ref[:] = jnp.dot(y1_ref[:], wBs[r - 1][:],
                            preferred_element_type=jnp.float32
                            ).astype(jnp.bfloat16)
        prep_layer1(r)

        def m_mid(k, acc, r=r):
            acc = layer2_block(k, acc)
            layer1_block(r, k, r + 1 if r < 2 else None)
            return acc
        acc = lax.fori_loop(0, _NB, m_mid, jnp.zeros((1, _C), jnp.float32),
                            unroll=4)
        outs.append(jnp.dot(acc * (1.0 / _N), lws[r - 1][:],
                            preferred_element_type=jnp.float32)
                    + lbs[r - 1][:])

    h1_ref[:] = jnp.dot(y1_ref[:], wBs[2][:],
                        preferred_element_type=jnp.float32).astype(jnp.bfloat16)
    acc = lax.fori_loop(0, _NB, layer2_block,
                        jnp.zeros((1, _C), jnp.float32), unroll=4)
    outs.append(jnp.dot(acc * (1.0 / _N), lws[2][:],
                        preferred_element_type=jnp.float32) + lbs[2][:])

    out_ref[:] = outs[0] + outs[1] + outs[2]


def kernel(x_0, x_1, x_2, laplacian_0, laplacian_1, laplacian_2,
           W0_l0, W1_l0, W2_l0, W0_l1, W1_l1, W2_l1,
           lin0_w, lin0_b, lin1_w, lin1_b, lin2_w, lin2_b):
    ncls = lin0_w.shape[1]
    hbm = pl.BlockSpec(memory_space=pltpu.MemorySpace.HBM)
    vmem = pl.BlockSpec(memory_space=pltpu.VMEM)
    out = pl.pallas_call(
        _body,
        out_shape=jax.ShapeDtypeStruct((1, ncls), jnp.float32),
        in_specs=[hbm, hbm, hbm] + [vmem] * 15,
        out_specs=vmem,
        scratch_shapes=[
            pltpu.VMEM((_N, _N), jnp.bfloat16),
            pltpu.VMEM((2, _BS, _N), jnp.float32),
            pltpu.VMEM((_N, _C), jnp.float32),
            pltpu.VMEM((_N, _C), jnp.bfloat16),
            pltpu.VMEM((_N, _C), jnp.bfloat16),
            pltpu.SemaphoreType.DMA((2,)),
        ],
        compiler_params=pltpu.CompilerParams(
            vmem_limit_bytes=62 * 1024 * 1024),
    )(laplacian_0, laplacian_1, laplacian_2, x_0, x_1, x_2,
      W0_l0, W0_l1, W1_l0, W1_l1, W2_l0, W2_l1,
      lin0_w, lin0_b.reshape(1, ncls), lin1_w, lin1_b.reshape(1, ncls),
      lin2_w, lin2_b.reshape(1, ncls))
    return out.reshape(-1)


# speedup vs baseline: 1.3366x; 1.0384x over previous
import jax
import jax.numpy as jnp
from jax import lax
from jax.experimental import pallas as pl
from jax.experimental.pallas import tpu as pltpu

_N = 4096
_BS = 256
_NB = _N // _BS
_C = 32
_NS = 4


def _dot16(a16, b16):
    return lax.dot_general(a16, b16, (((1,), (0,)), ((), ())),
                           preferred_element_type=jnp.float32)


def _body(L0, L1, L2, x0, x1, x2,
          w00, w01, w10, w11, w20, w21,
          lw0, lb0, lw1, lb1, lw2, lb2,
          out_ref, L16, sbuf, y1_ref, h0_ref, h1_ref, sem):
    Ls = (L0, L1, L2)
    xs = (x0, x1, x2)
    wAs = (w00, w10, w20)
    wBs = (w01, w11, w21)
    lws = (lw0, lw1, lw2)
    lbs = (lb0, lb1, lb2)

    def copy_blk(r, k, slot):
        return pltpu.make_async_copy(
            Ls[r].at[pl.ds(k * _BS, _BS), :], sbuf.at[slot], sem.at[slot])

    def prep_layer1(r):
        h0 = jnp.dot(xs[r][:], wAs[r][:], preferred_element_type=jnp.float32)
        h0_ref[:] = h0.astype(jnp.bfloat16)

    def layer1_block(r, k, next_r):
        slot = lax.rem(k, _NS)
        copy_blk(r, k, slot).wait()
        blk16 = sbuf[slot].astype(jnp.bfloat16)
        L16[pl.ds(k * _BS, _BS), :] = blk16
        y1_ref[pl.ds(k * _BS, _BS), :] = jnp.maximum(
            _dot16(blk16, h0_ref[:]), 0.0)

        @pl.when(k + _NS < _NB)
        def _():
            copy_blk(r, k + _NS, slot).start()

        if next_r is not None:
            @pl.when(k + _NS >= _NB)
            def _():
                copy_blk(next_r, k - (_NB - _NS), slot).start()

    def layer2_block(k, acc):
        y2 = jnp.maximum(_dot16(L16[pl.ds(k * _BS, _BS), :], h1_ref[:]), 0.0)
        return acc + jnp.sum(y2, axis=0, keepdims=True)

    for j in range(_NS):
        copy_blk(0, j, j).start()
    prep_layer1(0)

    def m0(k, c):
        layer1_block(0, k, 1)
        return c
    lax.fori_loop(0, _NB, m0, 0, unroll=4)

    outs = []
    for r in (1, 2):
        h1_ref[:] = jnp.dot(y1_ref[:], wBs[r - 1][:],
                            preferred_element_type=jnp.float32
                            ).astype(jnp.bfloat16)
        prep_layer1(r)

        def m_mid(k, acc, r=r):
            acc = layer2_block(k, acc)
            layer1_block(r, k, r + 1 if r < 2 else None)
            return acc
        acc = lax.fori_loop(0, _NB, m_mid, jnp.zeros((1, _C), jnp.float32),
                            unroll=4)
        outs.append(jnp.dot(acc * (1.0 / _N), lws[r - 1][:],
                            preferred_element_type=jnp.float32)
                    + lbs[r - 1][:])

    h1_ref[:] = jnp.dot(y1_ref[:], wBs[2][:],
                        preferred_element_type=jnp.float32).astype(jnp.bfloat16)
    acc = lax.fori_loop(0, _NB, layer2_block,
                        jnp.zeros((1, _C), jnp.float32), unroll=4)
    outs.append(jnp.dot(acc * (1.0 / _N), lws[2][:],
                        preferred_element_type=jnp.float32) + lbs[2][:])

    out_ref[:] = outs[0] + outs[1] + outs[2]


def kernel(x_0, x_1, x_2, laplacian_0, laplacian_1, laplacian_2,
           W0_l0, W1_l0, W2_l0, W0_l1, W1_l1, W2_l1,
           lin0_w, lin0_b, lin1_w, lin1_b, lin2_w, lin2_b):
    ncls = lin0_w.shape[1]
    hbm = pl.BlockSpec(memory_space=pltpu.MemorySpace.HBM)
    vmem = pl.BlockSpec(memory_space=pltpu.VMEM)
    out = pl.pallas_call(
        _body,
        out_shape=jax.ShapeDtypeStruct((1, ncls), jnp.float32),
        in_specs=[hbm, hbm, hbm] + [vmem] * 15,
        out_specs=vmem,
        scratch_shapes=[
            pltpu.VMEM((_N, _N), jnp.bfloat16),
            pltpu.VMEM((_NS, _BS, _N), jnp.float32),
            pltpu.VMEM((_N, _C), jnp.float32),
            pltpu.VMEM((_N, _C), jnp.bfloat16),
            pltpu.VMEM((_N, _C), jnp.bfloat16),
            pltpu.SemaphoreType.DMA((_NS,)),
        ],
        compiler_params=pltpu.CompilerParams(
            vmem_limit_bytes=62 * 1024 * 1024),
    )(laplacian_0, laplacian_1, laplacian_2, x_0, x_1, x_2,
      W0_l0, W0_l1, W1_l0, W1_l1, W2_l0, W2_l1,
      lin0_w, lin0_b.reshape(1, ncls), lin1_w, lin1_b.reshape(1, ncls),
      lin2_w, lin2_b.reshape(1, ncls))
    return out.reshape(-1)
